# baseline (speedup 1.0000x reference)
import functools

import jax
import jax.numpy as jnp
from jax import lax
from jax.experimental import pallas as pl
from jax.experimental.pallas import tpu as pltpu
from jax.experimental.pallas import tpu_sc as plsc

N = 10000
NP = 10240
NW = 32
NPW = NP // NW
E = 320000
EB = 128
EPAD = E + 8 * EB
RSP = 336
NRS = NP + RSP

_NEG = -1.0e30


def _seg_scan(vals, seg, iot, combine):
    for sh in (1, 2, 4, 8):
        pidx = jnp.maximum(iot - sh, 0)
        pv = vals.at[pidx].get(mode="promise_in_bounds")
        pd = seg.at[pidx].get(mode="promise_in_bounds")
        take = (pd == seg) & (iot >= sh)
        vals = jnp.where(take, combine(vals, pv), vals)
    return vals


def _make_gat_sc(dout, dc):
    C = dout // dc
    KS = dc // 16
    mesh = plsc.VectorSubcoreMesh(core_axis_name="c", subcore_axis_name="s",
                                  num_cores=2, num_subcores=16)
    out_type = [jax.ShapeDtypeStruct((C * NP, dc), jnp.float32),
                jax.ShapeDtypeStruct((NP,), jnp.float32)]
    NS = 3
    scratch = [
        pltpu.VMEM((NP,), jnp.float32),
        pltpu.VMEM((NP,), jnp.float32),
        pltpu.VMEM((RSP,), jnp.int32),
        pltpu.VMEM((EB + 16,), jnp.float32),
        pltpu.VMEM((NPW,), jnp.float32),
        pltpu.VMEM((NPW,), jnp.float32),
        pltpu.VMEM((NPW, dc), jnp.float32),
    ]
    scratch += [pltpu.VMEM((EB + 16,), jnp.int32) for _ in range(NS)]
    scratch += [pltpu.VMEM((EB + 16,), jnp.int32) for _ in range(NS)]
    scratch += [pltpu.VMEM((EB,), jnp.int32) for _ in range(NS)]
    scratch += [pltpu.VMEM((EB, dc), jnp.float32) for _ in range(NS)]
    scratch += [pltpu.SemaphoreType.DMA for _ in range(2 * NS)]

    def body(*refs):
        h_all = refs[0]
        asrc_h, adst_h, src_h, dst_h, rs_h = refs[1:6]
        S_all = refs[6]
        den_h = refs[7]
        r = list(refs[8:])
        asrc_t, adst_t, rs_t, wb, amax_t, den_t, acc = r[:7]
        srcb = r[7:7 + NS]
        dstb = r[7 + NS:7 + 2 * NS]
        gidx = r[7 + 2 * NS:7 + 3 * NS]
        rows = r[7 + 3 * NS:7 + 4 * NS]
        sem_i = r[7 + 4 * NS:7 + 5 * NS]
        sem_r = r[7 + 5 * NS:7 + 6 * NS]

        wid = lax.axis_index("s") * 2 + lax.axis_index("c")
        n0 = wid * NPW
        pltpu.sync_copy(asrc_h, asrc_t)
        pltpu.sync_copy(adst_h, adst_t)
        pltpu.sync_copy(rs_h.at[pl.ds(n0, RSP)], rs_t)
        e0 = rs_t[pl.ds(0, 16)][0]
        e1 = rs_t[pl.ds(NPW, 16)][0]
        ea = (e0 // EB) * EB
        nb = (e1 - ea + EB - 1) // EB
        iot = lax.iota(jnp.int32, 16)

        def init(i, _):
            amax_t[pl.ds(i * 16, 16)] = jnp.full((16,), _NEG, jnp.float32)
            den_t[pl.ds(i * 16, 16)] = jnp.zeros((16,), jnp.float32)
            return 0
        lax.fori_loop(0, NPW // 16, init, 0, unroll=2)

        def idx_copies(b, s, with_g):
            base = ea + b * EB
            ops = [(src_h.at[pl.ds(base, EB)], srcb[s].at[pl.ds(0, EB)]),
                   (dst_h.at[pl.ds(base, EB)], dstb[s].at[pl.ds(0, EB)])]
            if with_g:
                ops.append((src_h.at[pl.ds(base, EB)], gidx[s]))
            return ops

        def issue_idx(b, s, with_g):
            for sr, ds_ in idx_copies(b, s, with_g):
                pltpu.async_copy(sr, ds_, sem_i[s])

        def wait_idx(b, s, with_g):
            for sr, ds_ in idx_copies(b, s, with_g):
                pltpu.make_async_copy(sr, ds_, sem_i[s]).wait()

        def fix_gidx(s, off):
            for v in range(EB // 16):
                sl = pl.ds(v * 16, 16)
                gidx[s][sl] = gidx[s][sl] + off

        def issue_rows(s):
            pltpu.async_copy(h_all.at[gidx[s]], rows[s], sem_r[s])

        def wait_rows(s):
            pltpu.make_async_copy(h_all.at[gidx[s]], rows[s], sem_r[s]).wait()

        def alpha_sub(base, v, s):
            sl = pl.ds(v * 16, 16)
            sidx = srcb[s][sl]
            didx = dstb[s][sl]
            ev = base + v * 16 + iot
            valid = (ev >= e0) & (ev < e1)
            a = (plsc.load_gather(asrc_t, [sidx])
                 + plsc.load_gather(adst_t, [didx]))
            a = jnp.where(a >= 0.0, a, 0.2 * a)
            dl = jnp.clip(didx - n0, 0, NPW - 1)
            seg = jnp.where(valid, didx, -1)
            return a, dl, seg, valid

        def lastrun(seg, valid):
            nxt = seg.at[jnp.minimum(iot + 1, 15)].get(
                mode="promise_in_bounds")
            return valid & ((seg != nxt) | (iot == 15))

        def p1_compute(b, s):
            base = ea + b * EB
            for v in range(EB // 16):
                a, dl, seg, valid = alpha_sub(base, v, s)
                m = jnp.where(valid, a, _NEG)
                m = _seg_scan(m, seg, iot, jnp.maximum)
                wm = lastrun(seg, valid)
                cur = plsc.load_gather(amax_t, [dl])
                plsc.store_scatter(amax_t, [dl], jnp.maximum(cur, m), mask=wm)

        issue_idx(0, 0, False)
        def p1body(i, _):
            for j in range(2):
                b = 2 * i + j
                issue_idx(b + 1, 1 - j, False)
                wait_idx(b, j, False)
                p1_compute(b, j)
            return 0
        nb2 = 2 * ((nb + 1) // 2)
        lax.fori_loop(0, nb2 // 2, p1body, 0)
        wait_idx(nb2, 0, False)

        def p2_compute(b, s):
            base = ea + b * EB
            for v in range(EB // 16):
                a, dl, seg, valid = alpha_sub(base, v, s)
                am = plsc.load_gather(amax_t, [dl])
                w = jnp.where(valid, jnp.exp(a - am), 0.0)
                sm = _seg_scan(w, seg, iot, lambda x, y: x + y)
                wm = lastrun(seg, valid)
                plsc.addupdate_scatter(den_t, [dl], sm, mask=wm)

        issue_idx(0, 0, False)
        def p2body(i, _):
            for j in range(2):
                b = 2 * i + j
                issue_idx(b + 1, 1 - j, False)
                wait_idx(b, j, False)
                p2_compute(b, j)
            return 0
        lax.fori_loop(0, nb2 // 2, p2body, 0)
        wait_idx(nb2, 0, False)
        pltpu.sync_copy(den_t, den_h.at[pl.ds(n0, NPW)])

        nb3 = 3 * ((nb + 2) // 3)

        def zero(i, _):
            for k in range(KS):
                acc[i, pl.ds(k * 16, 16)] = jnp.zeros((16,), jnp.float32)
            return 0

        def mp_compute(b, s):
            base = ea + b * EB
            for v in range(EB // 16):
                a, dl, seg, valid = alpha_sub(base, v, s)
                am = plsc.load_gather(amax_t, [dl])
                wb[pl.ds(v * 16, 16)] = jnp.where(valid, jnp.exp(a - am), 0.0)

            def inner(i, _, s=s):
                dli = jnp.clip(dstb[s][pl.ds(i, 16)][0] - n0, 0, NPW - 1)
                wi = wb[pl.ds(i, 16)][0]
                for k in range(KS):
                    plsc.addupdate(acc.at[dli, pl.ds(k * 16, 16)],
                                   wi * rows[s][i, pl.ds(k * 16, 16)])
                return 0
            lax.fori_loop(0, EB, inner, 0, unroll=8)

        def chunk_body(cc, _):
            off = cc * NP
            lax.fori_loop(0, NPW, zero, 0, unroll=4)
            for s in range(NS):
                issue_idx(s, s, True)
            wait_idx(0, 0, True)
            fix_gidx(0, off)
            issue_rows(0)

            def mpbody(i, _):
                for j in range(NS):
                    b = NS * i + j
                    sn = (j + 1) % NS
                    wait_idx(b + 1, sn, True)
                    fix_gidx(sn, off)
                    issue_rows(sn)
                    wait_rows(j)
                    mp_compute(b, j)
                    issue_idx(b + NS, j, True)
                return 0
            lax.fori_loop(0, nb3 // 3, mpbody, 0)
            wait_idx(nb3 + 1, 1, True)
            wait_idx(nb3 + 2, 2, True)
            wait_rows(0)
            pltpu.sync_copy(acc, S_all.at[pl.ds(off + n0, NPW)])
            return 0
        lax.fori_loop(0, C, chunk_body, 0)

    return pl.kernel(
        body, out_type, mesh=mesh, scratch_types=scratch,
        compiler_params=pltpu.CompilerParams(needs_layout_passes=False),
        name=f"gat_sc_{dout}")


_BR = 512


def _first_tc(x, W, a_s, a_d):
    din, dout = W.shape

    def body(x_ref, w_ref, as_ref, ad_ref, h_ref, s_ref, d_ref):
        h = jnp.dot(x_ref[...], w_ref[...], preferred_element_type=jnp.float32)
        h_ref[...] = h
        s_ref[...] = jnp.sum(h * as_ref[...], axis=1)
        d_ref[...] = jnp.sum(h * ad_ref[...], axis=1)

    return pl.pallas_call(
        body,
        grid=(NP // _BR,),
        in_specs=[
            pl.BlockSpec((_BR, din), lambda i: (i, 0)),
            pl.BlockSpec((din, dout), lambda i: (0, 0)),
            pl.BlockSpec((1, dout), lambda i: (0, 0)),
            pl.BlockSpec((1, dout), lambda i: (0, 0)),
        ],
        out_specs=[
            pl.BlockSpec((_BR, dout), lambda i: (i, 0)),
            pl.BlockSpec((_BR,), lambda i: (i,)),
            pl.BlockSpec((_BR,), lambda i: (i,)),
        ],
        out_shape=[
            jax.ShapeDtypeStruct((NP, dout), jnp.float32),
            jax.ShapeDtypeStruct((NP,), jnp.float32),
            jax.ShapeDtypeStruct((NP,), jnp.float32),
        ],
    )(x, W, a_s.reshape(1, dout), a_d.reshape(1, dout))


def _mid_tc(S3, den, b, W, a_s, a_d, dc_out):
    Cin, _, dcin = S3.shape
    din, dout = W.shape
    Cout = dout // dc_out

    def body(s_ref, den_ref, b_ref, w_ref, as_ref, ad_ref,
             h_ref, so_ref, do_ref):
        inv = 1.0 / (den_ref[...].reshape(_BR, 1) + 1e-16)
        parts = [jnp.maximum(s_ref[c] * inv
                             + b_ref[0, c * dcin:(c + 1) * dcin], 0.0)
                 for c in range(Cin)]
        xb = jnp.concatenate(parts, axis=1)[:, :din]
        h = jnp.dot(xb, w_ref[...], preferred_element_type=jnp.float32)
        for c in range(Cout):
            h_ref[c] = h[:, c * dc_out:(c + 1) * dc_out]
        so_ref[...] = jnp.sum(h * as_ref[...], axis=1)
        do_ref[...] = jnp.sum(h * ad_ref[...], axis=1)

    out_shape = [jax.ShapeDtypeStruct((Cout, NP, dc_out), jnp.float32),
                 jax.ShapeDtypeStruct((NP,), jnp.float32),
                 jax.ShapeDtypeStruct((NP,), jnp.float32)]
    out_specs = [pl.BlockSpec((Cout, _BR, dc_out), lambda i: (0, i, 0)),
                 pl.BlockSpec((_BR,), lambda i: (i,)),
                 pl.BlockSpec((_BR,), lambda i: (i,))]
    in_specs = [
        pl.BlockSpec((Cin, _BR, dcin), lambda i: (0, i, 0)),
        pl.BlockSpec((_BR,), lambda i: (i,)),
        pl.BlockSpec((1, Cin * dcin), lambda i: (0, 0)),
        pl.BlockSpec((din, dout), lambda i: (0, 0)),
        pl.BlockSpec((1, dout), lambda i: (0, 0)),
        pl.BlockSpec((1, dout), lambda i: (0, 0)),
    ]
    bp = jnp.zeros((1, Cin * dcin), jnp.float32).at[0, :din].set(b)
    return pl.pallas_call(
        body, grid=(NP // _BR,), in_specs=in_specs, out_specs=out_specs,
        out_shape=out_shape,
    )(S3, den, bp, W, a_s.reshape(1, dout), a_d.reshape(1, dout))


def _head_tc(S3, den, b3, w1, b1, w2p, b2p):
    Cin, _, dcin = S3.shape
    dh = Cin * dcin
    nblk = NP // _BR

    def body(s_ref, den_ref, b_ref, w1_ref, b1_ref, w2_ref, b2_ref,
             out_ref, g_ref):
        i = pl.program_id(0)

        @pl.when(i == 0)
        def _():
            g_ref[...] = jnp.zeros_like(g_ref)

        inv = 1.0 / (den_ref[...].reshape(_BR, 1) + 1e-16)
        rows = i * _BR + lax.broadcasted_iota(jnp.int32, (_BR, 1), 0)
        rmask = rows < N
        parts = [jnp.maximum(s_ref[c] * inv
                             + b_ref[0, c * dcin:(c + 1) * dcin], 0.0)
                 for c in range(Cin)]
        xb = jnp.where(rmask, jnp.concatenate(parts, axis=1), 0.0)
        g_ref[...] = jnp.maximum(g_ref[...], jnp.max(xb, axis=0,
                                                     keepdims=True))

        @pl.when(i == nblk - 1)
        def _():
            g = g_ref[...]
            z = jnp.maximum(
                jnp.dot(g, w1_ref[...], preferred_element_type=jnp.float32)
                + b1_ref[...], 0.0)
            logits = jnp.dot(z, w2_ref[...],
                             preferred_element_type=jnp.float32) + b2_ref[...]
            mx = jnp.max(logits, axis=1, keepdims=True)
            sh = logits - mx
            lse = jnp.log(jnp.sum(jnp.exp(sh), axis=1, keepdims=True))
            out_ref[...] = sh - lse

    in_specs = [
        pl.BlockSpec((Cin, _BR, dcin), lambda i: (0, i, 0)),
        pl.BlockSpec((_BR,), lambda i: (i,)),
        pl.BlockSpec((1, dh), lambda i: (0, 0)),
        pl.BlockSpec((dh, 512), lambda i: (0, 0)),
        pl.BlockSpec((1, 512), lambda i: (0, 0)),
        pl.BlockSpec((512, 128), lambda i: (0, 0)),
        pl.BlockSpec((1, 128), lambda i: (0, 0)),
    ]
    out, _ = pl.pallas_call(
        body, grid=(nblk,),
        in_specs=in_specs,
        out_specs=[pl.BlockSpec((1, 128), lambda i: (0, 0)),
                   pl.BlockSpec((1, dh), lambda i: (0, 0))],
        out_shape=[jax.ShapeDtypeStruct((1, 128), jnp.float32),
                   jax.ShapeDtypeStruct((1, dh), jnp.float32)],
    )(S3, den, b3.reshape(1, dh), w1, b1.reshape(1, 512), w2p,
      b2p.reshape(1, 128))
    return out


_gat1 = _make_gat_sc(128, 128)
_gat2 = _make_gat_sc(256, 128)
_gat3 = _make_gat_sc(1024, 128)


def kernel(x, edge_index, edge_attr, W1, a_src1, a_dst1, b1,
           W2, a_src2, a_dst2, b2, W3, a_src3, a_dst3, b3,
           lin1_W, lin1_b, lin2_W, lin2_b):
    src = edge_index[0]
    dst = edge_index[1]
    order = jnp.argsort(dst)
    src_s = jnp.zeros((EPAD,), jnp.int32).at[:E].set(src[order])
    dst_sE = dst[order]
    dst_s = jnp.zeros((EPAD,), jnp.int32).at[:E].set(dst_sE)
    rs = jnp.searchsorted(dst_sE, jnp.arange(NRS, dtype=jnp.int32),
                          side='left').astype(jnp.int32)
    xp = jnp.zeros((NP, x.shape[1]), x.dtype).at[:N].set(x)

    W1p = jnp.zeros((W1.shape[0], 128), jnp.float32).at[:, :64].set(W1)
    as1p = jnp.zeros((128,), jnp.float32).at[:64].set(a_src1)
    ad1p = jnp.zeros((128,), jnp.float32).at[:64].set(a_dst1)
    h1, s1, d1 = _first_tc(xp, W1p, as1p, ad1p)
    S1, den1 = _gat1(h1, s1, d1, src_s, dst_s, rs)

    h2, s2, d2 = _mid_tc(S1.reshape(1, NP, 128), den1, b1,
                         W2, a_src2, a_dst2, 128)
    S2, den2 = _gat2(h2.reshape(2 * NP, 128), s2, d2, src_s, dst_s, rs)

    h3, s3, d3 = _mid_tc(S2.reshape(2, NP, 128), den2, b2,
                         W3, a_src3, a_dst3, 128)
    S3, den3 = _gat3(h3.reshape(8 * NP, 128), s3, d3, src_s, dst_s, rs)

    w2p = jnp.zeros((512, 128), jnp.float32).at[:, :40].set(lin2_W)
    b2p = jnp.full((128,), -1e30, jnp.float32).at[:40].set(lin2_b)
    logits = _head_tc(S3.reshape(8, NP, 128), den3, b3, lin1_W, lin1_b,
                      w2p, b2p)
    return logits[:, :40]


# speedup vs baseline: 2.5099x; 2.5099x over previous
import functools

import jax
import jax.numpy as jnp
from jax import lax
from jax.experimental import pallas as pl
from jax.experimental.pallas import tpu as pltpu
from jax.experimental.pallas import tpu_sc as plsc

N = 10000
NP = 10240
NW = 32
NPW = NP // NW
E = 320000
EB = 128
EPAD = E + 8 * EB
RSP = 336
NRS = NP + RSP

_NEG = -1.0e30


def _seg_scan(vals, seg, iot, combine):
    for sh in (1, 2, 4, 8):
        pidx = jnp.maximum(iot - sh, 0)
        pv = vals.at[pidx].get(mode="promise_in_bounds")
        pd = seg.at[pidx].get(mode="promise_in_bounds")
        take = (pd == seg) & (iot >= sh)
        vals = jnp.where(take, combine(vals, pv), vals)
    return vals


def _make_gat_sc(dout, dc):
    C = dout // dc
    KS = dc // 16
    mesh = plsc.VectorSubcoreMesh(core_axis_name="c", subcore_axis_name="s",
                                  num_cores=2, num_subcores=16)
    out_type = [jax.ShapeDtypeStruct((C * NP, dc), jnp.float32),
                jax.ShapeDtypeStruct((NP,), jnp.float32)]
    NS = 3
    scratch = [
        pltpu.VMEM((NP,), jnp.float32),
        pltpu.VMEM((NP,), jnp.float32),
        pltpu.VMEM((RSP,), jnp.int32),
        pltpu.VMEM((EB + 16,), jnp.float32),
        pltpu.VMEM((NPW,), jnp.float32),
        pltpu.VMEM((NPW,), jnp.float32),
        pltpu.VMEM((NPW, dc), jnp.float32),
    ]
    scratch += [pltpu.VMEM((EB + 16,), jnp.int32) for _ in range(NS)]
    scratch += [pltpu.VMEM((EB + 16,), jnp.int32) for _ in range(NS)]
    scratch += [pltpu.VMEM((EB,), jnp.int32) for _ in range(NS)]
    scratch += [pltpu.VMEM((EB, dc), jnp.float32) for _ in range(NS)]
    scratch += [pltpu.SemaphoreType.DMA for _ in range(2 * NS)]

    def body(*refs):
        h_all = refs[0]
        asrc_h, adst_h, src_h, dst_h, rs_h = refs[1:6]
        S_all = refs[6]
        den_h = refs[7]
        r = list(refs[8:])
        asrc_t, adst_t, rs_t, wb, amax_t, den_t, acc = r[:7]
        srcb = r[7:7 + NS]
        dstb = r[7 + NS:7 + 2 * NS]
        gidx = r[7 + 2 * NS:7 + 3 * NS]
        rows = r[7 + 3 * NS:7 + 4 * NS]
        sem_i = r[7 + 4 * NS:7 + 5 * NS]
        sem_r = r[7 + 5 * NS:7 + 6 * NS]

        wid = lax.axis_index("s") * 2 + lax.axis_index("c")
        n0 = wid * NPW
        pltpu.sync_copy(asrc_h, asrc_t)
        pltpu.sync_copy(adst_h, adst_t)
        pltpu.sync_copy(rs_h.at[pl.ds(n0, RSP)], rs_t)
        e0 = rs_t[pl.ds(0, 16)][0]
        e1 = rs_t[pl.ds(NPW, 16)][0]
        ea = (e0 // EB) * EB
        nb = (e1 - ea + EB - 1) // EB
        iot = lax.iota(jnp.int32, 16)

        def init(i, _):
            amax_t[pl.ds(i * 16, 16)] = jnp.full((16,), _NEG, jnp.float32)
            den_t[pl.ds(i * 16, 16)] = jnp.zeros((16,), jnp.float32)
            return 0
        lax.fori_loop(0, NPW // 16, init, 0, unroll=2)

        def idx_copies(b, s, with_g):
            base = ea + b * EB
            ops = [(src_h.at[pl.ds(base, EB)], srcb[s].at[pl.ds(0, EB)]),
                   (dst_h.at[pl.ds(base, EB)], dstb[s].at[pl.ds(0, EB)])]
            if with_g:
                ops.append((src_h.at[pl.ds(base, EB)], gidx[s]))
            return ops

        def issue_idx(b, s, with_g):
            for sr, ds_ in idx_copies(b, s, with_g):
                pltpu.async_copy(sr, ds_, sem_i[s])

        def wait_idx(b, s, with_g):
            for sr, ds_ in idx_copies(b, s, with_g):
                pltpu.make_async_copy(sr, ds_, sem_i[s]).wait()

        def fix_gidx(s, off):
            for v in range(EB // 16):
                sl = pl.ds(v * 16, 16)
                gidx[s][sl] = gidx[s][sl] + off

        def issue_rows(s):
            pltpu.async_copy(h_all.at[gidx[s]], rows[s], sem_r[s])

        def wait_rows(s):
            pltpu.make_async_copy(h_all.at[gidx[s]], rows[s], sem_r[s]).wait()

        def alpha_sub(base, v, s):
            sl = pl.ds(v * 16, 16)
            sidx = srcb[s][sl]
            didx = dstb[s][sl]
            ev = base + v * 16 + iot
            valid = (ev >= e0) & (ev < e1)
            a = (plsc.load_gather(asrc_t, [sidx])
                 + plsc.load_gather(adst_t, [didx]))
            a = jnp.where(a >= 0.0, a, 0.2 * a)
            dl = jnp.clip(didx - n0, 0, NPW - 1)
            seg = jnp.where(valid, didx, -1)
            return a, dl, seg, valid

        def lastrun(seg, valid):
            nxt = seg.at[jnp.minimum(iot + 1, 15)].get(
                mode="promise_in_bounds")
            return valid & ((seg != nxt) | (iot == 15))

        def p1_compute(b, s):
            base = ea + b * EB
            for v in range(EB // 16):
                a, dl, seg, valid = alpha_sub(base, v, s)
                m = jnp.where(valid, a, _NEG)
                m = _seg_scan(m, seg, iot, jnp.maximum)
                wm = lastrun(seg, valid)
                cur = plsc.load_gather(amax_t, [dl])
                plsc.store_scatter(amax_t, [dl], jnp.maximum(cur, m), mask=wm)

        issue_idx(0, 0, False)
        def p1body(i, _):
            for j in range(2):
                b = 2 * i + j
                issue_idx(b + 1, 1 - j, False)
                wait_idx(b, j, False)
                p1_compute(b, j)
            return 0
        nb2 = 2 * ((nb + 1) // 2)
        lax.fori_loop(0, nb2 // 2, p1body, 0)
        wait_idx(nb2, 0, False)

        def p2_compute(b, s):
            base = ea + b * EB
            for v in range(EB // 16):
                a, dl, seg, valid = alpha_sub(base, v, s)
                am = plsc.load_gather(amax_t, [dl])
                w = jnp.where(valid, jnp.exp(a - am), 0.0)
                sm = _seg_scan(w, seg, iot, lambda x, y: x + y)
                wm = lastrun(seg, valid)
                plsc.addupdate_scatter(den_t, [dl], sm, mask=wm)

        issue_idx(0, 0, False)
        def p2body(i, _):
            for j in range(2):
                b = 2 * i + j
                issue_idx(b + 1, 1 - j, False)
                wait_idx(b, j, False)
                p2_compute(b, j)
            return 0
        lax.fori_loop(0, nb2 // 2, p2body, 0)
        wait_idx(nb2, 0, False)
        pltpu.sync_copy(den_t, den_h.at[pl.ds(n0, NPW)])

        nb3 = 3 * ((nb + 2) // 3)

        def zero(i, _):
            for k in range(KS):
                acc[i, pl.ds(k * 16, 16)] = jnp.zeros((16,), jnp.float32)
            return 0

        def mp_compute(b, s):
            base = ea + b * EB
            for v in range(EB // 16):
                a, dl, seg, valid = alpha_sub(base, v, s)
                am = plsc.load_gather(amax_t, [dl])
                wb[pl.ds(v * 16, 16)] = jnp.where(valid, jnp.exp(a - am), 0.0)

            @plsc.parallel_loop(0, EB, 1, unroll=8)
            def inner(i, s=s):
                dli = jnp.clip(dstb[s][pl.ds(i, 16)][0] - n0, 0, NPW - 1)
                wi = wb[pl.ds(i, 16)][0]
                for k in range(KS):
                    plsc.addupdate(acc.at[dli, pl.ds(k * 16, 16)],
                                   wi * rows[s][i, pl.ds(k * 16, 16)])

        def chunk_body(cc, _):
            off = cc * NP
            lax.fori_loop(0, NPW, zero, 0, unroll=4)
            for s in range(NS):
                issue_idx(s, s, True)
            wait_idx(0, 0, True)
            fix_gidx(0, off)
            issue_rows(0)

            def mpbody(i, _):
                for j in range(NS):
                    b = NS * i + j
                    sn = (j + 1) % NS
                    wait_idx(b + 1, sn, True)
                    fix_gidx(sn, off)
                    issue_rows(sn)
                    wait_rows(j)
                    mp_compute(b, j)
                    issue_idx(b + NS, j, True)
                return 0
            lax.fori_loop(0, nb3 // 3, mpbody, 0)
            wait_idx(nb3 + 1, 1, True)
            wait_idx(nb3 + 2, 2, True)
            wait_rows(0)
            pltpu.sync_copy(acc, S_all.at[pl.ds(off + n0, NPW)])
            return 0
        lax.fori_loop(0, C, chunk_body, 0)

    return pl.kernel(
        body, out_type, mesh=mesh, scratch_types=scratch,
        compiler_params=pltpu.CompilerParams(needs_layout_passes=False),
        name=f"gat_sc_{dout}")


_BR = 512


def _first_tc(x, W, a_s, a_d):
    din, dout = W.shape

    def body(x_ref, w_ref, as_ref, ad_ref, h_ref, s_ref, d_ref):
        h = jnp.dot(x_ref[...], w_ref[...], preferred_element_type=jnp.float32)
        h_ref[...] = h
        s_ref[...] = jnp.sum(h * as_ref[...], axis=1)
        d_ref[...] = jnp.sum(h * ad_ref[...], axis=1)

    return pl.pallas_call(
        body,
        grid=(NP // _BR,),
        in_specs=[
            pl.BlockSpec((_BR, din), lambda i: (i, 0)),
            pl.BlockSpec((din, dout), lambda i: (0, 0)),
            pl.BlockSpec((1, dout), lambda i: (0, 0)),
            pl.BlockSpec((1, dout), lambda i: (0, 0)),
        ],
        out_specs=[
            pl.BlockSpec((_BR, dout), lambda i: (i, 0)),
            pl.BlockSpec((_BR,), lambda i: (i,)),
            pl.BlockSpec((_BR,), lambda i: (i,)),
        ],
        out_shape=[
            jax.ShapeDtypeStruct((NP, dout), jnp.float32),
            jax.ShapeDtypeStruct((NP,), jnp.float32),
            jax.ShapeDtypeStruct((NP,), jnp.float32),
        ],
    )(x, W, a_s.reshape(1, dout), a_d.reshape(1, dout))


def _mid_tc(S3, den, b, W, a_s, a_d, dc_out):
    Cin, _, dcin = S3.shape
    din, dout = W.shape
    Cout = dout // dc_out

    def body(s_ref, den_ref, b_ref, w_ref, as_ref, ad_ref,
             h_ref, so_ref, do_ref):
        inv = 1.0 / (den_ref[...].reshape(_BR, 1) + 1e-16)
        parts = [jnp.maximum(s_ref[c] * inv
                             + b_ref[0, c * dcin:(c + 1) * dcin], 0.0)
                 for c in range(Cin)]
        xb = jnp.concatenate(parts, axis=1)[:, :din]
        h = jnp.dot(xb, w_ref[...], preferred_element_type=jnp.float32)
        for c in range(Cout):
            h_ref[c] = h[:, c * dc_out:(c + 1) * dc_out]
        so_ref[...] = jnp.sum(h * as_ref[...], axis=1)
        do_ref[...] = jnp.sum(h * ad_ref[...], axis=1)

    out_shape = [jax.ShapeDtypeStruct((Cout, NP, dc_out), jnp.float32),
                 jax.ShapeDtypeStruct((NP,), jnp.float32),
                 jax.ShapeDtypeStruct((NP,), jnp.float32)]
    out_specs = [pl.BlockSpec((Cout, _BR, dc_out), lambda i: (0, i, 0)),
                 pl.BlockSpec((_BR,), lambda i: (i,)),
                 pl.BlockSpec((_BR,), lambda i: (i,))]
    in_specs = [
        pl.BlockSpec((Cin, _BR, dcin), lambda i: (0, i, 0)),
        pl.BlockSpec((_BR,), lambda i: (i,)),
        pl.BlockSpec((1, Cin * dcin), lambda i: (0, 0)),
        pl.BlockSpec((din, dout), lambda i: (0, 0)),
        pl.BlockSpec((1, dout), lambda i: (0, 0)),
        pl.BlockSpec((1, dout), lambda i: (0, 0)),
    ]
    bp = jnp.zeros((1, Cin * dcin), jnp.float32).at[0, :din].set(b)
    return pl.pallas_call(
        body, grid=(NP // _BR,), in_specs=in_specs, out_specs=out_specs,
        out_shape=out_shape,
    )(S3, den, bp, W, a_s.reshape(1, dout), a_d.reshape(1, dout))


def _head_tc(S3, den, b3, w1, b1, w2p, b2p):
    Cin, _, dcin = S3.shape
    dh = Cin * dcin
    nblk = NP // _BR

    def body(s_ref, den_ref, b_ref, w1_ref, b1_ref, w2_ref, b2_ref,
             out_ref, g_ref):
        i = pl.program_id(0)

        @pl.when(i == 0)
        def _():
            g_ref[...] = jnp.zeros_like(g_ref)

        inv = 1.0 / (den_ref[...].reshape(_BR, 1) + 1e-16)
        rows = i * _BR + lax.broadcasted_iota(jnp.int32, (_BR, 1), 0)
        rmask = rows < N
        parts = [jnp.maximum(s_ref[c] * inv
                             + b_ref[0, c * dcin:(c + 1) * dcin], 0.0)
                 for c in range(Cin)]
        xb = jnp.where(rmask, jnp.concatenate(parts, axis=1), 0.0)
        g_ref[...] = jnp.maximum(g_ref[...], jnp.max(xb, axis=0,
                                                     keepdims=True))

        @pl.when(i == nblk - 1)
        def _():
            g = g_ref[...]
            z = jnp.maximum(
                jnp.dot(g, w1_ref[...], preferred_element_type=jnp.float32)
                + b1_ref[...], 0.0)
            logits = jnp.dot(z, w2_ref[...],
                             preferred_element_type=jnp.float32) + b2_ref[...]
            mx = jnp.max(logits, axis=1, keepdims=True)
            sh = logits - mx
            lse = jnp.log(jnp.sum(jnp.exp(sh), axis=1, keepdims=True))
            out_ref[...] = sh - lse

    in_specs = [
        pl.BlockSpec((Cin, _BR, dcin), lambda i: (0, i, 0)),
        pl.BlockSpec((_BR,), lambda i: (i,)),
        pl.BlockSpec((1, dh), lambda i: (0, 0)),
        pl.BlockSpec((dh, 512), lambda i: (0, 0)),
        pl.BlockSpec((1, 512), lambda i: (0, 0)),
        pl.BlockSpec((512, 128), lambda i: (0, 0)),
        pl.BlockSpec((1, 128), lambda i: (0, 0)),
    ]
    out, _ = pl.pallas_call(
        body, grid=(nblk,),
        in_specs=in_specs,
        out_specs=[pl.BlockSpec((1, 128), lambda i: (0, 0)),
                   pl.BlockSpec((1, dh), lambda i: (0, 0))],
        out_shape=[jax.ShapeDtypeStruct((1, 128), jnp.float32),
                   jax.ShapeDtypeStruct((1, dh), jnp.float32)],
    )(S3, den, b3.reshape(1, dh), w1, b1.reshape(1, 512), w2p,
      b2p.reshape(1, 128))
    return out


_gat1 = _make_gat_sc(128, 128)
_gat2 = _make_gat_sc(256, 128)
_gat3 = _make_gat_sc(1024, 128)


def kernel(x, edge_index, edge_attr, W1, a_src1, a_dst1, b1,
           W2, a_src2, a_dst2, b2, W3, a_src3, a_dst3, b3,
           lin1_W, lin1_b, lin2_W, lin2_b):
    src = edge_index[0]
    dst = edge_index[1]
    order = jnp.argsort(dst)
    src_s = jnp.zeros((EPAD,), jnp.int32).at[:E].set(src[order])
    dst_sE = dst[order]
    dst_s = jnp.zeros((EPAD,), jnp.int32).at[:E].set(dst_sE)
    rs = jnp.searchsorted(dst_sE, jnp.arange(NRS, dtype=jnp.int32),
                          side='left').astype(jnp.int32)
    xp = jnp.zeros((NP, x.shape[1]), x.dtype).at[:N].set(x)

    W1p = jnp.zeros((W1.shape[0], 128), jnp.float32).at[:, :64].set(W1)
    as1p = jnp.zeros((128,), jnp.float32).at[:64].set(a_src1)
    ad1p = jnp.zeros((128,), jnp.float32).at[:64].set(a_dst1)
    h1, s1, d1 = _first_tc(xp, W1p, as1p, ad1p)
    S1, den1 = _gat1(h1, s1, d1, src_s, dst_s, rs)

    h2, s2, d2 = _mid_tc(S1.reshape(1, NP, 128), den1, b1,
                         W2, a_src2, a_dst2, 128)
    S2, den2 = _gat2(h2.reshape(2 * NP, 128), s2, d2, src_s, dst_s, rs)

    h3, s3, d3 = _mid_tc(S2.reshape(2, NP, 128), den2, b2,
                         W3, a_src3, a_dst3, 128)
    S3, den3 = _gat3(h3.reshape(8 * NP, 128), s3, d3, src_s, dst_s, rs)

    w2p = jnp.zeros((512, 128), jnp.float32).at[:, :40].set(lin2_W)
    b2p = jnp.full((128,), -1e30, jnp.float32).at[:40].set(lin2_b)
    logits = _head_tc(S3.reshape(8, NP, 128), den3, b3, lin1_W, lin1_b,
                      w2p, b2p)
    return logits[:, :40]


# speedup vs baseline: 3.3703x; 1.3428x over previous
import functools

import jax
import jax.numpy as jnp
from jax import lax
from jax.experimental import pallas as pl
from jax.experimental.pallas import tpu as pltpu
from jax.experimental.pallas import tpu_sc as plsc

N = 10000
NP = 10240
NW = 32
NPW = NP // NW
E = 320000
EB = 128
EPAD = E + 8 * EB
RSP = 48

_NEG = -1.0e30


def _seg_scan(vals, seg, iot, combine):
    for sh in (1, 2, 4, 8):
        pidx = jnp.maximum(iot - sh, 0)
        pv = vals.at[pidx].get(mode="promise_in_bounds")
        pd = seg.at[pidx].get(mode="promise_in_bounds")
        take = (pd == seg) & (iot >= sh)
        vals = jnp.where(take, combine(vals, pv), vals)
    return vals


def _make_gat_sc(dout, dc):
    C = dout // dc
    KS = dc // 16
    mesh = plsc.VectorSubcoreMesh(core_axis_name="c", subcore_axis_name="s",
                                  num_cores=2, num_subcores=16)
    out_type = [jax.ShapeDtypeStruct((C * NP, dc), jnp.float32),
                jax.ShapeDtypeStruct((NP,), jnp.float32)]
    NS = 3
    scratch = [
        pltpu.VMEM((NP,), jnp.float32),
        pltpu.VMEM((NP,), jnp.float32),
        pltpu.VMEM((RSP,), jnp.int32),
        pltpu.VMEM((EB + 16,), jnp.float32),
        pltpu.VMEM((NPW,), jnp.float32),
        pltpu.VMEM((NPW,), jnp.float32),
        pltpu.VMEM((NPW, dc), jnp.float32),
    ]
    scratch += [pltpu.VMEM((EB + 16,), jnp.int32) for _ in range(NS)]
    scratch += [pltpu.VMEM((EB + 16,), jnp.int32) for _ in range(NS)]
    scratch += [pltpu.VMEM((EB,), jnp.int32) for _ in range(NS)]
    scratch += [pltpu.VMEM((EB, dc), jnp.float32) for _ in range(NS)]
    scratch += [pltpu.SemaphoreType.DMA for _ in range(2 * NS)]

    def body(*refs):
        h_all = refs[0]
        asrc_h, adst_h, src_h, dst_h, rs_h = refs[1:6]
        S_all = refs[6]
        den_h = refs[7]
        r = list(refs[8:])
        asrc_t, adst_t, rs_t, wb, amax_t, den_t, acc = r[:7]
        srcb = r[7:7 + NS]
        dstb = r[7 + NS:7 + 2 * NS]
        gidx = r[7 + 2 * NS:7 + 3 * NS]
        rows = r[7 + 3 * NS:7 + 4 * NS]
        sem_i = r[7 + 4 * NS:7 + 5 * NS]
        sem_r = r[7 + 5 * NS:7 + 6 * NS]

        wid = lax.axis_index("s") * 2 + lax.axis_index("c")
        n0 = wid * NPW
        pltpu.sync_copy(asrc_h, asrc_t)
        pltpu.sync_copy(adst_h, adst_t)
        pltpu.sync_copy(rs_h, rs_t)
        widv = jnp.full((16,), wid, jnp.int32)
        e0 = plsc.load_gather(rs_t, [widv])[0]
        e1 = plsc.load_gather(rs_t, [widv + 1])[0]
        ea = (e0 // EB) * EB
        nb = (e1 - ea + EB - 1) // EB
        iot = lax.iota(jnp.int32, 16)

        def init(i, _):
            amax_t[pl.ds(i * 16, 16)] = jnp.full((16,), _NEG, jnp.float32)
            den_t[pl.ds(i * 16, 16)] = jnp.zeros((16,), jnp.float32)
            return 0
        lax.fori_loop(0, NPW // 16, init, 0, unroll=2)

        def idx_copies(b, s, with_g):
            base = ea + b * EB
            ops = [(src_h.at[pl.ds(base, EB)], srcb[s].at[pl.ds(0, EB)]),
                   (dst_h.at[pl.ds(base, EB)], dstb[s].at[pl.ds(0, EB)])]
            if with_g:
                ops.append((src_h.at[pl.ds(base, EB)], gidx[s]))
            return ops

        def issue_idx(b, s, with_g):
            for sr, ds_ in idx_copies(b, s, with_g):
                pltpu.async_copy(sr, ds_, sem_i[s])

        def wait_idx(b, s, with_g):
            for sr, ds_ in idx_copies(b, s, with_g):
                pltpu.make_async_copy(sr, ds_, sem_i[s]).wait()

        def fix_gidx(s, off):
            for v in range(EB // 16):
                sl = pl.ds(v * 16, 16)
                gidx[s][sl] = gidx[s][sl] + off

        def issue_rows(s):
            pltpu.async_copy(h_all.at[gidx[s]], rows[s], sem_r[s])

        def wait_rows(s):
            pltpu.make_async_copy(h_all.at[gidx[s]], rows[s], sem_r[s]).wait()

        def alpha_sub(base, v, s):
            sl = pl.ds(v * 16, 16)
            sidx = srcb[s][sl]
            didx = dstb[s][sl]
            ev = base + v * 16 + iot
            valid = (ev >= e0) & (ev < e1)
            a = (plsc.load_gather(asrc_t, [sidx])
                 + plsc.load_gather(adst_t, [didx]))
            a = jnp.where(a >= 0.0, a, 0.2 * a)
            dl = jnp.clip(didx - n0, 0, NPW - 1)
            seg = jnp.where(valid, didx, -1)
            return a, dl, seg, valid

        def lastrun(seg, valid):
            nxt = seg.at[jnp.minimum(iot + 1, 15)].get(
                mode="promise_in_bounds")
            return valid & ((seg != nxt) | (iot == 15))

        def p1_compute(b, s):
            base = ea + b * EB
            for v in range(EB // 16):
                a, dl, seg, valid = alpha_sub(base, v, s)
                m = jnp.where(valid, a, _NEG)
                m = _seg_scan(m, seg, iot, jnp.maximum)
                wm = lastrun(seg, valid)
                cur = plsc.load_gather(amax_t, [dl])
                plsc.store_scatter(amax_t, [dl], jnp.maximum(cur, m), mask=wm)

        issue_idx(0, 0, False)
        def p1body(i, _):
            for j in range(2):
                b = 2 * i + j
                issue_idx(b + 1, 1 - j, False)
                wait_idx(b, j, False)
                p1_compute(b, j)
            return 0
        nb2 = 2 * ((nb + 1) // 2)
        lax.fori_loop(0, nb2 // 2, p1body, 0)
        wait_idx(nb2, 0, False)

        def p2_compute(b, s):
            base = ea + b * EB
            for v in range(EB // 16):
                a, dl, seg, valid = alpha_sub(base, v, s)
                am = plsc.load_gather(amax_t, [dl])
                w = jnp.where(valid, jnp.exp(a - am), 0.0)
                sm = _seg_scan(w, seg, iot, lambda x, y: x + y)
                wm = lastrun(seg, valid)
                plsc.addupdate_scatter(den_t, [dl], sm, mask=wm)

        issue_idx(0, 0, False)
        def p2body(i, _):
            for j in range(2):
                b = 2 * i + j
                issue_idx(b + 1, 1 - j, False)
                wait_idx(b, j, False)
                p2_compute(b, j)
            return 0
        lax.fori_loop(0, nb2 // 2, p2body, 0)
        wait_idx(nb2, 0, False)
        pltpu.sync_copy(den_t, den_h.at[pl.ds(n0, NPW)])

        nb3 = 3 * ((nb + 2) // 3)

        def zero(i, _):
            for k in range(KS):
                acc[i, pl.ds(k * 16, 16)] = jnp.zeros((16,), jnp.float32)
            return 0

        def mp_compute(b, s):
            base = ea + b * EB
            for v in range(EB // 16):
                a, dl, seg, valid = alpha_sub(base, v, s)
                am = plsc.load_gather(amax_t, [dl])
                wb[pl.ds(v * 16, 16)] = jnp.where(valid, jnp.exp(a - am), 0.0)

            @plsc.parallel_loop(0, EB, 1, unroll=8)
            def inner(i, s=s):
                dli = jnp.clip(dstb[s][pl.ds(i, 16)][0] - n0, 0, NPW - 1)
                wi = wb[pl.ds(i, 16)][0]
                for k in range(KS):
                    plsc.addupdate(acc.at[dli, pl.ds(k * 16, 16)],
                                   wi * rows[s][i, pl.ds(k * 16, 16)])

        def chunk_body(cc, _):
            off = cc * NP
            lax.fori_loop(0, NPW, zero, 0, unroll=4)
            for s in range(NS):
                issue_idx(s, s, True)
            wait_idx(0, 0, True)
            fix_gidx(0, off)
            issue_rows(0)

            def mpbody(i, _):
                for j in range(NS):
                    b = NS * i + j
                    sn = (j + 1) % NS
                    wait_idx(b + 1, sn, True)
                    fix_gidx(sn, off)
                    issue_rows(sn)
                    wait_rows(j)
                    mp_compute(b, j)
                    issue_idx(b + NS, j, True)
                return 0
            lax.fori_loop(0, nb3 // 3, mpbody, 0)
            wait_idx(nb3 + 1, 1, True)
            wait_idx(nb3 + 2, 2, True)
            wait_rows(0)
            pltpu.sync_copy(acc, S_all.at[pl.ds(off + n0, NPW)])
            return 0
        lax.fori_loop(0, C, chunk_body, 0)

    return pl.kernel(
        body, out_type, mesh=mesh, scratch_types=scratch,
        compiler_params=pltpu.CompilerParams(needs_layout_passes=False),
        name=f"gat_sc_{dout}")


_BR = 512


def _first_tc(x, W, a_s, a_d):
    din, dout = W.shape

    def body(x_ref, w_ref, as_ref, ad_ref, h_ref, s_ref, d_ref):
        h = jnp.dot(x_ref[...], w_ref[...], preferred_element_type=jnp.float32)
        h_ref[...] = h
        s_ref[...] = jnp.sum(h * as_ref[...], axis=1)
        d_ref[...] = jnp.sum(h * ad_ref[...], axis=1)

    return pl.pallas_call(
        body,
        grid=(NP // _BR,),
        in_specs=[
            pl.BlockSpec((_BR, din), lambda i: (i, 0)),
            pl.BlockSpec((din, dout), lambda i: (0, 0)),
            pl.BlockSpec((1, dout), lambda i: (0, 0)),
            pl.BlockSpec((1, dout), lambda i: (0, 0)),
        ],
        out_specs=[
            pl.BlockSpec((_BR, dout), lambda i: (i, 0)),
            pl.BlockSpec((_BR,), lambda i: (i,)),
            pl.BlockSpec((_BR,), lambda i: (i,)),
        ],
        out_shape=[
            jax.ShapeDtypeStruct((NP, dout), jnp.float32),
            jax.ShapeDtypeStruct((NP,), jnp.float32),
            jax.ShapeDtypeStruct((NP,), jnp.float32),
        ],
    )(x, W, a_s.reshape(1, dout), a_d.reshape(1, dout))


def _mid_tc(S3, den, b, W, a_s, a_d, dc_out):
    Cin, _, dcin = S3.shape
    din, dout = W.shape
    Cout = dout // dc_out

    def body(s_ref, den_ref, b_ref, w_ref, as_ref, ad_ref,
             h_ref, so_ref, do_ref):
        inv = 1.0 / (den_ref[...].reshape(_BR, 1) + 1e-16)
        parts = [jnp.maximum(s_ref[c] * inv
                             + b_ref[0, c * dcin:(c + 1) * dcin], 0.0)
                 for c in range(Cin)]
        xb = jnp.concatenate(parts, axis=1)[:, :din]
        h = jnp.dot(xb, w_ref[...], preferred_element_type=jnp.float32)
        for c in range(Cout):
            h_ref[c] = h[:, c * dc_out:(c + 1) * dc_out]
        so_ref[...] = jnp.sum(h * as_ref[...], axis=1)
        do_ref[...] = jnp.sum(h * ad_ref[...], axis=1)

    out_shape = [jax.ShapeDtypeStruct((Cout, NP, dc_out), jnp.float32),
                 jax.ShapeDtypeStruct((NP,), jnp.float32),
                 jax.ShapeDtypeStruct((NP,), jnp.float32)]
    out_specs = [pl.BlockSpec((Cout, _BR, dc_out), lambda i: (0, i, 0)),
                 pl.BlockSpec((_BR,), lambda i: (i,)),
                 pl.BlockSpec((_BR,), lambda i: (i,))]
    in_specs = [
        pl.BlockSpec((Cin, _BR, dcin), lambda i: (0, i, 0)),
        pl.BlockSpec((_BR,), lambda i: (i,)),
        pl.BlockSpec((1, Cin * dcin), lambda i: (0, 0)),
        pl.BlockSpec((din, dout), lambda i: (0, 0)),
        pl.BlockSpec((1, dout), lambda i: (0, 0)),
        pl.BlockSpec((1, dout), lambda i: (0, 0)),
    ]
    bp = jnp.zeros((1, Cin * dcin), jnp.float32).at[0, :din].set(b)
    return pl.pallas_call(
        body, grid=(NP // _BR,), in_specs=in_specs, out_specs=out_specs,
        out_shape=out_shape,
    )(S3, den, bp, W, a_s.reshape(1, dout), a_d.reshape(1, dout))


def _head_tc(S3, den, b3, w1, b1, w2p, b2p):
    Cin, _, dcin = S3.shape
    dh = Cin * dcin
    nblk = NP // _BR

    def body(s_ref, den_ref, b_ref, w1_ref, b1_ref, w2_ref, b2_ref,
             out_ref, g_ref):
        i = pl.program_id(0)

        @pl.when(i == 0)
        def _():
            g_ref[...] = jnp.zeros_like(g_ref)

        inv = 1.0 / (den_ref[...].reshape(_BR, 1) + 1e-16)
        rows = i * _BR + lax.broadcasted_iota(jnp.int32, (_BR, 1), 0)
        rmask = rows < N
        parts = [jnp.maximum(s_ref[c] * inv
                             + b_ref[0, c * dcin:(c + 1) * dcin], 0.0)
                 for c in range(Cin)]
        xb = jnp.where(rmask, jnp.concatenate(parts, axis=1), 0.0)
        g_ref[...] = jnp.maximum(g_ref[...], jnp.max(xb, axis=0,
                                                     keepdims=True))

        @pl.when(i == nblk - 1)
        def _():
            g = g_ref[...]
            z = jnp.maximum(
                jnp.dot(g, w1_ref[...], preferred_element_type=jnp.float32)
                + b1_ref[...], 0.0)
            logits = jnp.dot(z, w2_ref[...],
                             preferred_element_type=jnp.float32) + b2_ref[...]
            mx = jnp.max(logits, axis=1, keepdims=True)
            sh = logits - mx
            lse = jnp.log(jnp.sum(jnp.exp(sh), axis=1, keepdims=True))
            out_ref[...] = sh - lse

    in_specs = [
        pl.BlockSpec((Cin, _BR, dcin), lambda i: (0, i, 0)),
        pl.BlockSpec((_BR,), lambda i: (i,)),
        pl.BlockSpec((1, dh), lambda i: (0, 0)),
        pl.BlockSpec((dh, 512), lambda i: (0, 0)),
        pl.BlockSpec((1, 512), lambda i: (0, 0)),
        pl.BlockSpec((512, 128), lambda i: (0, 0)),
        pl.BlockSpec((1, 128), lambda i: (0, 0)),
    ]
    out, _ = pl.pallas_call(
        body, grid=(nblk,),
        in_specs=in_specs,
        out_specs=[pl.BlockSpec((1, 128), lambda i: (0, 0)),
                   pl.BlockSpec((1, dh), lambda i: (0, 0))],
        out_shape=[jax.ShapeDtypeStruct((1, 128), jnp.float32),
                   jax.ShapeDtypeStruct((1, dh), jnp.float32)],
    )(S3, den, b3.reshape(1, dh), w1, b1.reshape(1, 512), w2p,
      b2p.reshape(1, 128))
    return out


_gat1 = _make_gat_sc(128, 128)
_gat2 = _make_gat_sc(256, 128)
_gat3 = _make_gat_sc(1024, 128)


def kernel(x, edge_index, edge_attr, W1, a_src1, a_dst1, b1,
           W2, a_src2, a_dst2, b2, W3, a_src3, a_dst3, b3,
           lin1_W, lin1_b, lin2_W, lin2_b):
    src = edge_index[0]
    dst = edge_index[1]
    dst_sE, src_sE = lax.sort((dst, src), num_keys=1, is_stable=False)
    src_s = jnp.zeros((EPAD,), jnp.int32).at[:E].set(src_sE)
    dst_s = jnp.zeros((EPAD,), jnp.int32).at[:E].set(dst_sE)
    qs = jnp.arange(RSP, dtype=jnp.int32) * NPW
    rs = jnp.searchsorted(dst_sE, qs, side='left').astype(jnp.int32)
    xp = jnp.zeros((NP, x.shape[1]), x.dtype).at[:N].set(x)

    W1p = jnp.zeros((W1.shape[0], 128), jnp.float32).at[:, :64].set(W1)
    as1p = jnp.zeros((128,), jnp.float32).at[:64].set(a_src1)
    ad1p = jnp.zeros((128,), jnp.float32).at[:64].set(a_dst1)
    h1, s1, d1 = _first_tc(xp, W1p, as1p, ad1p)
    S1, den1 = _gat1(h1, s1, d1, src_s, dst_s, rs)

    h2, s2, d2 = _mid_tc(S1.reshape(1, NP, 128), den1, b1,
                         W2, a_src2, a_dst2, 128)
    S2, den2 = _gat2(h2.reshape(2 * NP, 128), s2, d2, src_s, dst_s, rs)

    h3, s3, d3 = _mid_tc(S2.reshape(2, NP, 128), den2, b2,
                         W3, a_src3, a_dst3, 128)
    S3, den3 = _gat3(h3.reshape(8 * NP, 128), s3, d3, src_s, dst_s, rs)

    w2p = jnp.zeros((512, 128), jnp.float32).at[:, :40].set(lin2_W)
    b2p = jnp.full((128,), -1e30, jnp.float32).at[:40].set(lin2_b)
    logits = _head_tc(S3.reshape(8, NP, 128), den3, b3, lin1_W, lin1_b,
                      w2p, b2p)
    return logits[:, :40]
